# Initial kernel scaffold; baseline (speedup 1.0000x reference)
import functools

import jax
import jax.numpy as jnp
from jax import lax
from jax.experimental import pallas as pl
from jax.experimental.pallas import tpu as pltpu
from jax.experimental.pallas import tpu_sc as plsc

VOCAB = 1_000_000
DIM = 64

NC = 2
NS = 16
NW = NC * NS

IDXW = 128
G = 8
BLOCK = G * IDXW


def _emb_body(n_blocks, table_hbm, idx_hbm, out_hbm, idx_v, rows_v, gsem):
    wid = lax.axis_index("s") * NC + lax.axis_index("c")
    base = wid * (n_blocks * BLOCK)

    @pl.loop(0, n_blocks)
    def _block(i):
        off = base + i * BLOCK
        pltpu.sync_copy(idx_hbm.at[pl.ds(off, BLOCK)], idx_v)
        descs = []
        for j in range(G):
            descs.append(
                pltpu.async_copy(
                    table_hbm.at[idx_v.at[pl.ds(j * IDXW, IDXW)]],
                    rows_v.at[pl.ds(j * IDXW, IDXW)],
                    gsem,
                )
            )
        for d in descs:
            d.wait()
        pltpu.sync_copy(rows_v, out_hbm.at[pl.ds(off, BLOCK)])


@functools.partial(jax.jit, static_argnames=("n_blocks",))
def _emb(weight, idx_flat, n_blocks):
    mesh = plsc.VectorSubcoreMesh(
        core_axis_name="c", subcore_axis_name="s", num_cores=NC, num_subcores=NS
    )
    b_total = idx_flat.shape[0]
    return pl.kernel(
        functools.partial(_emb_body, n_blocks),
        out_type=jax.ShapeDtypeStruct((b_total, DIM), jnp.float32),
        mesh=mesh,
        scratch_types=[
            pltpu.VMEM((BLOCK,), jnp.int32),
            pltpu.VMEM((BLOCK, DIM), jnp.float32),
            pltpu.SemaphoreType.DMA,
        ],
    )(weight, idx_flat)


def kernel(x, weight):
    b, s = x.shape
    total = b * s
    assert total % (NW * BLOCK) == 0
    n_blocks = total // (NW * BLOCK)
    idx_flat = x.reshape(total).astype(jnp.int32)
    out = _emb(weight, idx_flat, n_blocks)
    return out.reshape(b, s, DIM)


# speedup vs baseline: 1.8432x; 1.8432x over previous
import functools

import jax
import jax.numpy as jnp
from jax import lax
from jax.experimental import pallas as pl
from jax.experimental.pallas import tpu as pltpu
from jax.experimental.pallas import tpu_sc as plsc

VOCAB = 1_000_000
DIM = 64

NC = 2
NS = 16
NW = NC * NS

IDXW = 128
G = 8
BLOCK = G * IDXW


def _emb_body(n_blocks, table_hbm, idx_hbm, out_hbm, idx_v, rows_v, gsem):
    wid = lax.axis_index("s") * NC + lax.axis_index("c")
    base = wid * (n_blocks * BLOCK)

    @pl.loop(0, n_blocks)
    def _block(i):
        off = base + i * BLOCK
        pltpu.sync_copy(idx_hbm.at[pl.ds(off, BLOCK)], idx_v)
        descs = []
        for j in range(G):
            descs.append(
                pltpu.async_copy(
                    table_hbm.at[idx_v.at[pl.ds(j * IDXW, IDXW)]],
                    rows_v.at[pl.ds(j * IDXW, IDXW)],
                    gsem,
                )
            )
        for d in descs:
            d.wait()
        pltpu.sync_copy(rows_v, out_hbm.at[pl.ds(off, BLOCK)])


@functools.partial(jax.jit, static_argnames=("n_blocks",))
def _emb(weight, idx_flat, n_blocks):
    mesh = plsc.VectorSubcoreMesh(
        core_axis_name="c", subcore_axis_name="s", num_cores=NC, num_subcores=NS
    )
    b_total = idx_flat.shape[0]
    return pl.kernel(
        functools.partial(_emb_body, n_blocks),
        out_type=jax.ShapeDtypeStruct((b_total, DIM), jnp.float32),
        mesh=mesh,
        scratch_types=[
            pltpu.VMEM((BLOCK,), jnp.int32),
            pltpu.VMEM((BLOCK, DIM), jnp.float32),
            pltpu.SemaphoreType.DMA,
        ],
        compiler_params=pltpu.CompilerParams(use_tc_tiling_on_sc=False),
    )(weight, idx_flat)


def kernel(x, weight):
    b, s = x.shape
    total = b * s
    assert total % (NW * BLOCK) == 0
    n_blocks = total // (NW * BLOCK)
    idx_flat = x.reshape(total).astype(jnp.int32)
    out = _emb(weight, idx_flat, n_blocks)
    return out.reshape(b, s, DIM)


# speedup vs baseline: 1.8744x; 1.0169x over previous
import functools

import jax
import jax.numpy as jnp
from jax import lax
from jax.experimental import pallas as pl
from jax.experimental.pallas import tpu as pltpu
from jax.experimental.pallas import tpu_sc as plsc

VOCAB = 1_000_000
DIM = 64

NC = 2
NS = 16
NW = NC * NS

IDXW = 128
G = 5
BLOCK = G * IDXW


def _emb_body(n_blocks, table, idx_hbm, out, idx_all, rows0, rows1,
              gsem0, gsem1, osem0, osem1):
    wid = lax.axis_index("s") * NC + lax.axis_index("c")
    bpw = n_blocks * BLOCK
    base = wid * bpw
    pltpu.sync_copy(idx_hbm.at[pl.ds(base, bpw)], idx_all)

    def fire_gathers(g, rows_buf, sem):
        for j in range(G):
            pltpu.async_copy(
                table.at[idx_all.at[pl.ds(g * BLOCK + j * IDXW, IDXW)]],
                rows_buf.at[pl.ds(j * IDXW, IDXW)],
                sem,
            )

    def drain_gathers(rows_buf, sem):
        for j in range(G):
            pltpu.make_async_copy(
                table.at[pl.ds(0, IDXW)],
                rows_buf.at[pl.ds(j * IDXW, IDXW)],
                sem,
            ).wait()

    def fire_out(g, rows_buf, sem):
        pltpu.async_copy(rows_buf, out.at[pl.ds(base + g * BLOCK, BLOCK)], sem)

    def drain_out(rows_buf, sem):
        pltpu.make_async_copy(rows_buf, out.at[pl.ds(base, BLOCK)], sem).wait()

    fire_gathers(0, rows0, gsem0)

    @pl.loop(0, n_blocks // 2)
    def _pair(i):
        g0 = 2 * i
        g1 = g0 + 1

        @pl.when(i > 0)
        def _():
            drain_out(rows1, osem1)

        fire_gathers(g1, rows1, gsem1)
        drain_gathers(rows0, gsem0)
        fire_out(g0, rows0, osem0)
        drain_out(rows0, osem0)

        @pl.when(g0 + 2 < n_blocks)
        def _():
            fire_gathers(g0 + 2, rows0, gsem0)

        drain_gathers(rows1, gsem1)
        fire_out(g1, rows1, osem1)

    drain_out(rows1, osem1)


@functools.partial(jax.jit, static_argnames=("n_blocks",))
def _emb(weight, idx_flat, n_blocks):
    mesh = plsc.VectorSubcoreMesh(
        core_axis_name="c", subcore_axis_name="s", num_cores=NC, num_subcores=NS
    )
    b_total = idx_flat.shape[0]
    return pl.kernel(
        functools.partial(_emb_body, n_blocks),
        out_type=jax.ShapeDtypeStruct((b_total, DIM), jnp.float32),
        mesh=mesh,
        scratch_types=[
            pltpu.VMEM((n_blocks * BLOCK,), jnp.int32),
            pltpu.VMEM((BLOCK, DIM), jnp.float32),
            pltpu.VMEM((BLOCK, DIM), jnp.float32),
            pltpu.SemaphoreType.DMA,
            pltpu.SemaphoreType.DMA,
            pltpu.SemaphoreType.DMA,
            pltpu.SemaphoreType.DMA,
        ],
        compiler_params=pltpu.CompilerParams(use_tc_tiling_on_sc=False),
    )(weight, idx_flat)


def kernel(x, weight):
    b, s = x.shape
    total = b * s
    assert total % (NW * BLOCK) == 0
    n_blocks = total // (NW * BLOCK)
    assert n_blocks % 2 == 0
    idx_flat = x.reshape(total).astype(jnp.int32)
    out = _emb(weight, idx_flat, n_blocks)
    return out.reshape(b, s, DIM)
